# baseline (speedup 1.0000x reference)
import jax
import jax.numpy as jnp
from jax.experimental import pallas as pl
from jax.experimental.pallas import tpu as pltpu

_BLOCK_ROWS = 8192


def _linear_kernel(x_ref, wt_ref, o_ref):
    o_ref[...] = jnp.dot(x_ref[...].astype(jnp.bfloat16),
                         wt_ref[...].astype(jnp.bfloat16),
                         preferred_element_type=jnp.float32)


def kernel(x, W):
    B, T, D = x.shape
    rows = B * T
    xf = x.reshape(rows, D)
    wt = W.T
    block = min(_BLOCK_ROWS, rows)
    grid = pl.cdiv(rows, block)
    y = pl.pallas_call(
        _linear_kernel,
        grid=(grid,),
        in_specs=[
            pl.BlockSpec((block, D), lambda i: (i, 0)),
            pl.BlockSpec((D, D), lambda i: (0, 0)),
        ],
        out_specs=pl.BlockSpec((block, D), lambda i: (i, 0)),
        out_shape=jax.ShapeDtypeStruct((rows, D), x.dtype),
        compiler_params=pltpu.CompilerParams(
            dimension_semantics=("parallel",),
        ),
    )(xf, wt)
    return y.reshape(B, T, D)


# speedup vs baseline: 1.2886x; 1.2886x over previous
import jax
import jax.numpy as jnp
from jax.experimental import pallas as pl
from jax.experimental.pallas import tpu as pltpu

_BLOCK_ROWS = 16384


def _linear_kernel(x_ref, w_ref, o_ref):
    o_ref[...] = jax.lax.dot_general(
        x_ref[...].astype(jnp.bfloat16),
        w_ref[...].astype(jnp.bfloat16),
        dimension_numbers=(((1,), (1,)), ((), ())),
        preferred_element_type=jnp.float32,
    )


def kernel(x, W):
    B, T, D = x.shape
    rows = B * T
    xf = x.reshape(rows, D)
    block = min(_BLOCK_ROWS, rows)
    grid = pl.cdiv(rows, block)
    y = pl.pallas_call(
        _linear_kernel,
        grid=(grid,),
        in_specs=[
            pl.BlockSpec((block, D), lambda i: (i, 0)),
            pl.BlockSpec((D, D), lambda i: (0, 0)),
        ],
        out_specs=pl.BlockSpec((block, D), lambda i: (i, 0)),
        out_shape=jax.ShapeDtypeStruct((rows, D), x.dtype),
        compiler_params=pltpu.CompilerParams(
            dimension_semantics=("arbitrary",),
        ),
    )(xf, W)
    return y.reshape(B, T, D)
